# Initial kernel scaffold; baseline (speedup 1.0000x reference)
import jax
import jax.numpy as jnp
from jax.experimental import pallas as pl


def kernel(all_rois, gt_boxes):
    raise NotImplementedError("write your pallas kernel here")


# speedup vs baseline: 6.8539x; 6.8539x over previous
import jax
import jax.numpy as jnp
from jax.experimental import pallas as pl
from jax.experimental.pallas import tpu as pltpu

_N = 20000
_G = 64
_NE = _N + _G
_ROWS = 157
_LANES = 128
_PAD = _ROWS * _LANES
_NCLS = 81
_FG = 32
_NROI = 128
_BG_PID = 5532.0


def _body(x1_ref, y1_ref, x2_ref, y2_ref, gt_ref,
          sel_ref, cls_ref, tgt_ref, inw_ref, outw_ref, pid_ref):
    x1 = x1_ref[...]
    y1 = y1_ref[...]
    x2 = x2_ref[...]
    y2 = y2_ref[...]
    area_b = (x2 - x1 + 1.0) * (y2 - y1 + 1.0)

    def gt_step(g, carry):
        mx, ag = carry
        gx1 = gt_ref[g, 0]
        gy1 = gt_ref[g, 1]
        gx2 = gt_ref[g, 2]
        gy2 = gt_ref[g, 3]
        area_q = (gx2 - gx1 + 1.0) * (gy2 - gy1 + 1.0)
        iw = jnp.minimum(x2, gx2) - jnp.maximum(x1, gx1) + 1.0
        ih = jnp.minimum(y2, gy2) - jnp.maximum(y1, gy1) + 1.0
        iw = jnp.maximum(iw, 0.0)
        ih = jnp.maximum(ih, 0.0)
        inter = iw * ih
        ua = area_b + area_q - inter
        iou = inter / ua
        upd = iou > mx
        mx = jnp.where(upd, iou, mx)
        ag = jnp.where(upd, g, ag)
        return mx, ag

    mx0 = jnp.full((_ROWS, _LANES), -1.0, jnp.float32)
    ag0 = jnp.zeros((_ROWS, _LANES), jnp.int32)
    mx, ag = jax.lax.fori_loop(0, _G, gt_step, (mx0, ag0))

    lin = (jax.lax.broadcasted_iota(jnp.int32, (_ROWS, _LANES), 0) * _LANES
           + jax.lax.broadcasted_iota(jnp.int32, (_ROWS, _LANES), 1))
    valid = lin < _NE
    fg = valid & (mx >= 0.5)
    bg = valid & (mx < 0.5) & (mx >= 0.0)

    li = jax.lax.broadcasted_iota(jnp.int32, (_LANES, _LANES), 0)
    ji = jax.lax.broadcasted_iota(jnp.int32, (_LANES, _LANES), 1)
    utri = (li <= ji).astype(jnp.float32)
    ri = jax.lax.broadcasted_iota(jnp.int32, (_ROWS, _ROWS), 0)
    ci = jax.lax.broadcasted_iota(jnp.int32, (_ROWS, _ROWS), 1)
    ltri = (ci < ri).astype(jnp.float32)

    def ranks(m):
        rowcum = jax.lax.dot(m, utri, preferred_element_type=jnp.float32)
        prev = jax.lax.dot(ltri, m, preferred_element_type=jnp.float32)
        off = jnp.sum(prev, axis=1, keepdims=True)
        return rowcum + off

    fgf = fg.astype(jnp.float32)
    bgf = bg.astype(jnp.float32)
    rfg = ranks(fgf)
    rbg = ranks(bgf)
    big = jnp.int32(100000)
    slot = jnp.where(fg & (rfg <= float(_FG)), rfg - 1.0, 1.0 * big)
    slot = jnp.where(bg & (rbg <= float(_NROI - _FG)), rbg + float(_FG - 1), slot)
    slot = slot.astype(jnp.int32)
    bg_total = jnp.minimum(jnp.sum(bgf), float(_NROI - _FG))

    agf = ag.astype(jnp.float32)

    def slot_step(j, acc):
        eq = (slot == j).astype(jnp.float32)
        s0 = jnp.sum(eq * x1)
        s1 = jnp.sum(eq * y1)
        s2 = jnp.sum(eq * x2)
        s3 = jnp.sum(eq * y2)
        s4 = jnp.sum(eq * agf)
        rowm = (jax.lax.broadcasted_iota(jnp.int32, (_NROI, 8), 0) == j)
        col = jax.lax.broadcasted_iota(jnp.int32, (_NROI, 8), 1)
        vals = (jnp.where(col == 0, s0, 0.0) + jnp.where(col == 1, s1, 0.0)
                + jnp.where(col == 2, s2, 0.0) + jnp.where(col == 3, s3, 0.0)
                + jnp.where(col == 4, s4, 0.0))
        return acc + jnp.where(rowm, vals, 0.0)

    acc = jax.lax.fori_loop(0, _NROI, slot_step,
                            jnp.zeros((_NROI, 8), jnp.float32))

    linf = lin == 0
    r0x1 = jnp.sum(jnp.where(linf, x1, 0.0))
    r0y1 = jnp.sum(jnp.where(linf, y1, 0.0))
    r0x2 = jnp.sum(jnp.where(linf, x2, 0.0))
    r0y2 = jnp.sum(jnp.where(linf, y2, 0.0))
    r0ag = jnp.sum(jnp.where(linf, agf, 0.0))
    row8 = jax.lax.broadcasted_iota(jnp.int32, (_NROI, 8), 0)
    col8 = jax.lax.broadcasted_iota(jnp.int32, (_NROI, 8), 1)
    empty = row8.astype(jnp.float32) >= (float(_FG) + bg_total)
    r0vals = (jnp.where(col8 == 0, r0x1, 0.0) + jnp.where(col8 == 1, r0y1, 0.0)
              + jnp.where(col8 == 2, r0x2, 0.0) + jnp.where(col8 == 3, r0y2, 0.0)
              + jnp.where(col8 == 4, r0ag, 0.0))
    acc = jnp.where(empty, r0vals, acc)

    asg = acc[:, 4:5]

    def ggt(g, c):
        gfv = g.astype(jnp.float32)
        m = asg == gfv
        vals = (jnp.where(col8 == 0, gt_ref[g, 0], 0.0)
                + jnp.where(col8 == 1, gt_ref[g, 1], 0.0)
                + jnp.where(col8 == 2, gt_ref[g, 2], 0.0)
                + jnp.where(col8 == 3, gt_ref[g, 3], 0.0)
                + jnp.where(col8 == 4, gt_ref[g, 4], 0.0)
                + jnp.where(col8 == 5, gt_ref[g, 5], 0.0))
        return c + jnp.where(m, vals, 0.0)

    gtacc = jax.lax.fori_loop(0, _G, ggt, jnp.zeros((_NROI, 8), jnp.float32))

    ex1 = acc[:, 0:1]
    ey1 = acc[:, 1:2]
    ex2 = acc[:, 2:3]
    ey2 = acc[:, 3:4]
    gx1 = gtacc[:, 0:1]
    gy1 = gtacc[:, 1:2]
    gx2 = gtacc[:, 2:3]
    gy2 = gtacc[:, 3:4]
    glab = gtacc[:, 4:5]
    gpid = gtacc[:, 5:6]

    ex_w = ex2 - ex1 + 1.0
    ex_h = ey2 - ey1 + 1.0
    ex_cx = ex1 + 0.5 * ex_w
    ex_cy = ey1 + 0.5 * ex_h
    gt_w = gx2 - gx1 + 1.0
    gt_h = gy2 - gy1 + 1.0
    gt_cx = gx1 + 0.5 * gt_w
    gt_cy = gy1 + 0.5 * gt_h
    dx = (gt_cx - ex_cx) / ex_w
    dy = (gt_cy - ex_cy) / ex_h
    dw = jnp.log(gt_w / ex_w)
    dh = jnp.log(gt_h / ex_h)
    dxn = (dx - 0.0) / 0.1
    dyn_ = (dy - 0.0) / 0.1
    dwn = (dw - 0.0) / 0.2
    dhn = (dh - 0.0) / 0.2

    jr = jax.lax.broadcasted_iota(jnp.int32, (_NROI, 1), 0)
    isfg = jr < _FG
    lab = jnp.where(isfg, glab, 0.0)
    clsf = jnp.round(lab)
    pidf = jnp.where(isfg, jnp.round(gpid), _BG_PID)

    cidx = jax.lax.broadcasted_iota(jnp.int32, (_NROI, 4 * _NCLS), 1)
    cls_i = clsf.astype(jnp.int32)
    rel = cidx - 4 * cls_i
    fgm = cls_i > 0
    tvals = (jnp.where(rel == 0, dxn, 0.0) + jnp.where(rel == 1, dyn_, 0.0)
             + jnp.where(rel == 2, dwn, 0.0) + jnp.where(rel == 3, dhn, 0.0))
    tgt_ref[...] = jnp.where(fgm, tvals, 0.0)
    inw = jnp.where(fgm & (rel >= 0) & (rel <= 3), 1.0, 0.0)
    inw_ref[...] = inw
    outw_ref[...] = inw

    sel_ref[...] = acc
    cls_ref[...] = clsf
    pid_ref[...] = pidf


def _run(x1, y1, x2, y2, gt):
    return pl.pallas_call(
        _body,
        out_shape=[
            jax.ShapeDtypeStruct((_NROI, 8), jnp.float32),
            jax.ShapeDtypeStruct((_NROI, 1), jnp.float32),
            jax.ShapeDtypeStruct((_NROI, 4 * _NCLS), jnp.float32),
            jax.ShapeDtypeStruct((_NROI, 4 * _NCLS), jnp.float32),
            jax.ShapeDtypeStruct((_NROI, 4 * _NCLS), jnp.float32),
            jax.ShapeDtypeStruct((_NROI, 1), jnp.float32),
        ],
        in_specs=[
            pl.BlockSpec(memory_space=pltpu.VMEM),
            pl.BlockSpec(memory_space=pltpu.VMEM),
            pl.BlockSpec(memory_space=pltpu.VMEM),
            pl.BlockSpec(memory_space=pltpu.VMEM),
            pl.BlockSpec(memory_space=pltpu.SMEM),
        ],
    )(x1, y1, x2, y2, gt)


@jax.jit
def kernel(all_rois, gt_boxes):
    coords = jnp.concatenate([all_rois[:, 1:5], gt_boxes[:, :4]], axis=0)
    coords = jnp.pad(coords, ((0, _PAD - _NE), (0, 0)))
    x1 = coords[:, 0].reshape(_ROWS, _LANES)
    y1 = coords[:, 1].reshape(_ROWS, _LANES)
    x2 = coords[:, 2].reshape(_ROWS, _LANES)
    y2 = coords[:, 3].reshape(_ROWS, _LANES)
    sel, clsf, tgt, inw, outw, pidf = _run(x1, y1, x2, y2, gt_boxes)
    rois = jnp.concatenate([jnp.zeros((_NROI, 1), jnp.float32), sel[:, :4]], axis=1)
    clss = clsf[:, 0].astype(jnp.int32)
    pid = pidf[:, 0].astype(jnp.int32)
    return (rois, clss, tgt, inw, outw, pid)
